# baseline (speedup 1.0000x reference)
import functools

import jax
import jax.numpy as jnp
from jax import lax
from jax.experimental import pallas as pl
from jax.experimental.pallas import tpu as pltpu
from jax.experimental.pallas import tpu_sc as plsc

_BATCH = 16384
_NUM_FIELDS = 26
_FIELD_SIZE = 100000
_TOTAL_ROWS = _NUM_FIELDS * _FIELD_SIZE
_NC, _NS, _L = 2, 16, 16
_NW = _NC * _NS
_BPW = _BATCH // _NW
_CHUNKS = _BPW // _L
_N = _NUM_FIELDS * _BPW

_XROWS = 32
_WPAD = ((_TOTAL_ROWS + 1023) // 1024) * 1024


def _body(x_hbm, w_hbm, b_hbm, out_hbm, idx_v, gat_v, out_v, bias_v, sem):
    wid = lax.axis_index("s") * _NC + lax.axis_index("c")
    base = wid * _BPW

    bias_v[...] = jnp.zeros((_L,), jnp.float32)
    pltpu.sync_copy(b_hbm, bias_v.at[pl.ds(0, 1)])

    for f in range(_NUM_FIELDS):
        pltpu.async_copy(
            x_hbm.at[pl.ds(f * _BATCH + base, _BPW)],
            idx_v.at[pl.ds(f * _BPW, _BPW)],
            sem,
        )
    for f in range(_NUM_FIELDS):
        pltpu.make_async_copy(
            x_hbm.at[pl.ds(f * _BATCH + base, _BPW)],
            idx_v.at[pl.ds(f * _BPW, _BPW)],
            sem,
        ).wait()

    pltpu.async_copy(w_hbm.at[idx_v], gat_v, sem).wait()

    bias_s = jnp.sum(bias_v[...])

    def reduce_chunk(c, carry):
        cb = c * _L
        acc = jnp.zeros((_L,), jnp.float32)
        for f in range(_NUM_FIELDS):
            acc = acc + gat_v[pl.ds(f * _BPW + cb, _L)]
        out_v[pl.ds(cb, _L)] = acc + bias_s
        return carry

    lax.fori_loop(0, _CHUNKS, reduce_chunk, 0)

    pltpu.sync_copy(out_v, out_hbm.at[pl.ds(base, _BPW)])


@functools.cache
def _build():
    mesh = plsc.VectorSubcoreMesh(core_axis_name="c", subcore_axis_name="s")
    return pl.kernel(
        _body,
        out_type=jax.ShapeDtypeStruct((_BATCH,), jnp.float32),
        mesh=mesh,
        scratch_types=[
            pltpu.VMEM((_N,), jnp.int32),
            pltpu.VMEM((_N,), jnp.float32),
            pltpu.VMEM((_BPW,), jnp.float32),
            pltpu.VMEM((_L,), jnp.float32),
            pltpu.SemaphoreType.DMA,
        ],
        compiler_params=pltpu.CompilerParams(needs_layout_passes=False),
    )


def kernel(x, fc_weight, bias):
    offs = jnp.arange(_NUM_FIELDS, dtype=jnp.int32) * _FIELD_SIZE
    xt = jnp.pad((x + offs[None, :]).T, ((0, _XROWS - _NUM_FIELDS), (0, 0)))
    x_flat = xt.reshape(_XROWS * _BATCH)
    out = _inner()(x_flat, fc_weight.reshape(_TOTAL_ROWS), bias)
    return out.reshape(_BATCH, 1)


@functools.cache
def _inner():
    from jax.experimental import layout as jax_layout

    devs = [d for d in jax.devices() if d.platform == "tpu"] or jax.devices()
    fmt = jax_layout.Format(
        jax_layout.Layout((0,), tiling=((128,),)),
        jax.sharding.SingleDeviceSharding(devs[0]),
    )
    return jax.jit(
        lambda x_flat, w_flat, b: _build()(x_flat, w_flat, b),
        in_shardings=(None, fmt, None),
    )


# speedup vs baseline: 2.7312x; 2.7312x over previous
import functools

import jax
import jax.numpy as jnp
from jax import lax
from jax.experimental import pallas as pl
from jax.experimental.pallas import tpu as pltpu
from jax.experimental.pallas import tpu_sc as plsc

_BATCH = 16384
_NUM_FIELDS = 26
_FIELD_SIZE = 100000
_TOTAL_ROWS = _NUM_FIELDS * _FIELD_SIZE
_NC, _NS, _L = 2, 16, 16
_NW = _NC * _NS
_BPW = _BATCH // _NW
_CHUNKS = _BPW // _L
_N = _NUM_FIELDS * _BPW

_XROWS = 32
_WPAD = ((_TOTAL_ROWS + 1023) // 1024) * 1024


_FH = _NUM_FIELDS // 2
_NH = _FH * _BPW


def _body(x_hbm, w_hbm, b_hbm, out_hbm,
          idx_a, idx_b, gat_a, gat_b, out_v, bias_v,
          sem_a, sem_b, sg_a, sg_b):
    wid = lax.axis_index("s") * _NC + lax.axis_index("c")
    base = wid * _BPW

    bias_v[...] = jnp.zeros((_L,), jnp.float32)
    pltpu.sync_copy(b_hbm, bias_v.at[pl.ds(0, 1)])

    def stage(f, dst, sem):
        return pltpu.make_async_copy(
            x_hbm.at[pl.ds(f * _BATCH + base, _BPW)],
            dst.at[pl.ds((f % _FH) * _BPW, _BPW)],
            sem,
        )

    for f in range(_FH):
        stage(f, idx_a, sem_a).start()
    for f in range(_FH):
        stage(_FH + f, idx_b, sem_b).start()

    for f in range(_FH):
        stage(f, idx_a, sem_a).wait()
    pltpu.async_copy(w_hbm.at[idx_a], gat_a, sg_a)
    for f in range(_FH):
        stage(_FH + f, idx_b, sem_b).wait()
    pltpu.async_copy(w_hbm.at[idx_b], gat_b, sg_b)

    bias_s = jnp.sum(bias_v[...])

    pltpu.make_async_copy(w_hbm.at[idx_a], gat_a, sg_a).wait()

    def reduce_a(c, carry):
        cb = c * _L
        acc = gat_a[pl.ds(cb, _L)]
        for f in range(1, _FH):
            acc = acc + gat_a[pl.ds(f * _BPW + cb, _L)]
        out_v[pl.ds(cb, _L)] = acc + bias_s
        return carry

    lax.fori_loop(0, _CHUNKS, reduce_a, 0)

    pltpu.make_async_copy(w_hbm.at[idx_b], gat_b, sg_b).wait()

    def reduce_b(c, carry):
        cb = c * _L
        acc = gat_b[pl.ds(cb, _L)]
        for f in range(1, _FH):
            acc = acc + gat_b[pl.ds(f * _BPW + cb, _L)]
        out_v[pl.ds(cb, _L)] = out_v[pl.ds(cb, _L)] + acc
        return carry

    lax.fori_loop(0, _CHUNKS, reduce_b, 0)

    pltpu.sync_copy(out_v, out_hbm.at[pl.ds(base, _BPW)])


@functools.cache
def _build():
    mesh = plsc.VectorSubcoreMesh(core_axis_name="c", subcore_axis_name="s")
    return pl.kernel(
        _body,
        out_type=jax.ShapeDtypeStruct((_BATCH,), jnp.float32),
        mesh=mesh,
        scratch_types=[
            pltpu.VMEM((_NH,), jnp.int32),
            pltpu.VMEM((_NH,), jnp.int32),
            pltpu.VMEM((_NH,), jnp.float32),
            pltpu.VMEM((_NH,), jnp.float32),
            pltpu.VMEM((_BPW,), jnp.float32),
            pltpu.VMEM((_L,), jnp.float32),
            pltpu.SemaphoreType.DMA,
            pltpu.SemaphoreType.DMA,
            pltpu.SemaphoreType.DMA,
            pltpu.SemaphoreType.DMA,
        ],
        compiler_params=pltpu.CompilerParams(needs_layout_passes=False),
    )


def kernel(x, fc_weight, bias):
    offs = jnp.arange(_NUM_FIELDS, dtype=jnp.int32) * _FIELD_SIZE
    xt = jnp.pad((x + offs[None, :]).T, ((0, _XROWS - _NUM_FIELDS), (0, 0)))
    x_flat = xt.reshape(_XROWS * _BATCH)
    wp = jnp.pad(fc_weight, ((0, _WPAD - _TOTAL_ROWS), (0, 0)))
    w_flat = wp.reshape(_WPAD)
    out = _build()(x_flat, w_flat, bias)
    return out.reshape(_BATCH, 1)


# speedup vs baseline: 2.7535x; 1.0081x over previous
import functools

import jax
import jax.numpy as jnp
from jax import lax
from jax.experimental import pallas as pl
from jax.experimental.pallas import tpu as pltpu
from jax.experimental.pallas import tpu_sc as plsc

_BATCH = 16384
_NUM_FIELDS = 26
_FIELD_SIZE = 100000
_TOTAL_ROWS = _NUM_FIELDS * _FIELD_SIZE
_NC, _NS, _L = 2, 16, 16
_NW = _NC * _NS
_BPW = _BATCH // _NW
_CHUNKS = _BPW // _L
_N = _NUM_FIELDS * _BPW

_XROWS = 32
_WPAD = ((_TOTAL_ROWS + 1023) // 1024) * 1024


_FH = _NUM_FIELDS // 2
_NH = _FH * _BPW


def _body(x_hbm, w_hbm, b_hbm, out_hbm,
          idx_a, idx_b, gat_a, gat_b, out_v, bias_v,
          sem_a, sem_b, sg_a, sg_b):
    wid = lax.axis_index("s") * _NC + lax.axis_index("c")
    base = wid * _BPW

    def stage(f, dst, sem):
        return pltpu.make_async_copy(
            x_hbm.at[pl.ds(f * _BATCH + base, _BPW)],
            dst.at[pl.ds((f % _FH) * _BPW, _BPW)],
            sem,
        )

    for f in range(_FH):
        stage(f, idx_a, sem_a).start()
    for f in range(_FH):
        stage(_FH + f, idx_b, sem_b).start()

    bias_v[...] = jnp.zeros((_L,), jnp.float32)
    pltpu.sync_copy(b_hbm, bias_v.at[pl.ds(0, 1)])

    for f in range(_FH):
        stage(f, idx_a, sem_a).wait()
    pltpu.async_copy(w_hbm.at[idx_a], gat_a, sg_a)
    for f in range(_FH):
        stage(_FH + f, idx_b, sem_b).wait()
    pltpu.async_copy(w_hbm.at[idx_b], gat_b, sg_b)

    bias_s = jnp.sum(bias_v[...])

    pltpu.make_async_copy(w_hbm.at[idx_a], gat_a, sg_a).wait()

    def reduce_a(c, carry):
        cb = c * _L
        acc = gat_a[pl.ds(cb, _L)]
        for f in range(1, _FH):
            acc = acc + gat_a[pl.ds(f * _BPW + cb, _L)]
        out_v[pl.ds(cb, _L)] = acc + bias_s
        return carry

    lax.fori_loop(0, _CHUNKS, reduce_a, 0)

    pltpu.make_async_copy(w_hbm.at[idx_b], gat_b, sg_b).wait()

    def reduce_b(c, carry):
        cb = c * _L
        acc = gat_b[pl.ds(cb, _L)]
        for f in range(1, _FH):
            acc = acc + gat_b[pl.ds(f * _BPW + cb, _L)]
        out_v[pl.ds(cb, _L)] = out_v[pl.ds(cb, _L)] + acc
        return carry

    lax.fori_loop(0, _CHUNKS, reduce_b, 0)

    pltpu.sync_copy(out_v, out_hbm.at[pl.ds(base, _BPW)])


@functools.cache
def _build():
    mesh = plsc.VectorSubcoreMesh(core_axis_name="c", subcore_axis_name="s")
    return pl.kernel(
        _body,
        out_type=jax.ShapeDtypeStruct((_BATCH,), jnp.float32),
        mesh=mesh,
        scratch_types=[
            pltpu.VMEM((_NH,), jnp.int32),
            pltpu.VMEM((_NH,), jnp.int32),
            pltpu.VMEM((_NH,), jnp.float32),
            pltpu.VMEM((_NH,), jnp.float32),
            pltpu.VMEM((_BPW,), jnp.float32),
            pltpu.VMEM((_L,), jnp.float32),
            pltpu.SemaphoreType.DMA,
            pltpu.SemaphoreType.DMA,
            pltpu.SemaphoreType.DMA,
            pltpu.SemaphoreType.DMA,
        ],
        compiler_params=pltpu.CompilerParams(needs_layout_passes=False),
    )


def kernel(x, fc_weight, bias):
    offs = jnp.arange(_NUM_FIELDS, dtype=jnp.int32) * _FIELD_SIZE
    xt = jnp.pad((x + offs[None, :]).T, ((0, _XROWS - _NUM_FIELDS), (0, 0)))
    x_flat = xt.reshape(_XROWS * _BATCH)
    wp = jnp.pad(fc_weight, ((0, _WPAD - _TOTAL_ROWS), (0, 0)))
    w_flat = wp.reshape(_WPAD)
    out = _build()(x_flat, w_flat, bias)
    return out.reshape(_BATCH, 1)
